# Initial kernel scaffold; baseline (speedup 1.0000x reference)
import jax
import jax.numpy as jnp
from jax.experimental import pallas as pl


def kernel(x, edge_index, W_l1, b_l1, W_r1, g1, be1, W_l2, b_l2, W_r2, g2, be2):
    raise NotImplementedError("write your pallas kernel here")


# speedup vs baseline: 7.6455x; 7.6455x over previous
import functools

import jax
import jax.numpy as jnp
from jax import lax
from jax.experimental import pallas as pl
from jax.experimental.pallas import tpu as pltpu
from jax.experimental.pallas import tpu_sc as plsc

_N = 10000
_D = 128
_E = 320000
_EPS = 1e-5

_NC = 2
_NS = 16
_NW = _NC * _NS
_CHUNK = 128
_NCHUNK = -(-_E // (_NW * _CHUNK))
_E_PAD = _NW * _NCHUNK * _CHUNK
_N_ACC = _N + 112
_RPT = _N_ACC // _NS

_RB = 1000
_NB = _N // _RB


def _mesh():
  return plsc.VectorSubcoreMesh(core_axis_name="c", subcore_axis_name="s")


def _agg_body(x_hbm, src_hbm, dst_hbm, zf_hbm, p_hbm, srcv, dstv, gbuf, accs,
              sem):
  cid = lax.axis_index("c")
  sid = lax.axis_index("s")
  wid = sid * _NC + cid
  r0 = sid * _RPT
  pltpu.sync_copy(zf_hbm.at[pl.ds(r0, _RPT)], accs.at[pl.ds(r0, _RPT)])
  pltpu.sync_copy(src_hbm.at[wid], srcv)
  pltpu.sync_copy(dst_hbm.at[wid], dstv)
  plsc.subcore_barrier()

  def step(j, carry):
    pltpu.async_copy(x_hbm.at[srcv.at[j]], gbuf, sem).wait()
    pltpu.sync_copy(gbuf, accs.at[dstv.at[j]], add=True)
    return carry

  lax.fori_loop(0, _NCHUNK, step, 0)
  plsc.subcore_barrier()
  pltpu.sync_copy(accs.at[pl.ds(r0, _RPT)], p_hbm.at[cid, pl.ds(r0, _RPT)])


_agg = pl.kernel(
    _agg_body,
    mesh=_mesh(),
    out_type=[jax.ShapeDtypeStruct((_NC, _N_ACC, _D), jnp.float32)],
    scratch_types=[
        pltpu.VMEM((_NCHUNK, _CHUNK), jnp.int32),
        pltpu.VMEM((_NCHUNK, _CHUNK), jnp.int32),
        pltpu.VMEM((_CHUNK, _D), jnp.float32),
        pltpu.VMEM_SHARED((_N_ACC, _D), jnp.float32),
        pltpu.SemaphoreType.DMA,
    ],
)


def _count_body(dst_hbm, zf_hbm, ones_hbm, cnt_hbm, dstv, onesv, cnts):
  cid = lax.axis_index("c")
  sid = lax.axis_index("s")
  wid = sid * _NC + cid
  r0 = sid * _RPT
  pltpu.sync_copy(zf_hbm.at[pl.ds(r0, _RPT)], cnts.at[pl.ds(r0, _RPT)])
  pltpu.sync_copy(ones_hbm, onesv)
  pltpu.sync_copy(dst_hbm.at[wid], dstv)
  plsc.subcore_barrier()

  def step(j, carry):
    pltpu.sync_copy(onesv, cnts.at[dstv.at[j]], add=True)
    return carry

  lax.fori_loop(0, _NCHUNK, step, 0)
  plsc.subcore_barrier()
  pltpu.sync_copy(cnts.at[pl.ds(r0, _RPT)], cnt_hbm.at[cid, pl.ds(r0, _RPT)])


_count = pl.kernel(
    _count_body,
    mesh=_mesh(),
    out_type=[jax.ShapeDtypeStruct((_NC, _N_ACC, _D), jnp.float32)],
    scratch_types=[
        pltpu.VMEM((_NCHUNK, _CHUNK), jnp.int32),
        pltpu.VMEM((_CHUNK, _D), jnp.float32),
        pltpu.VMEM_SHARED((_N_ACC, _D), jnp.float32),
    ],
)


def _dense_body(p_ref, c_ref, x_ref, wl_ref, bl_ref, wr_ref, u_ref, st_ref):
  agg = p_ref[0] + p_ref[1]
  cnt = jnp.maximum(c_ref[0][:, 0:1] + c_ref[1][:, 0:1], 1.0)
  agg = agg / cnt
  u = (lax.dot_general(agg, wl_ref[...], (((1,), (1,)), ((), ())),
                       preferred_element_type=jnp.float32)
       + bl_ref[...]
       + lax.dot_general(x_ref[...], wr_ref[...], (((1,), (1,)), ((), ())),
                         preferred_element_type=jnp.float32))
  u_ref[...] = u

  @pl.when(pl.program_id(0) == 0)
  def _():
    st_ref[...] = jnp.zeros_like(st_ref)

  st_ref[...] += jnp.concatenate(
      [jnp.sum(u, axis=0, keepdims=True),
       jnp.sum(u * u, axis=0, keepdims=True)], axis=0)


def _dense(p, cnt, x, w_l, b_l, w_r):
  return pl.pallas_call(
      _dense_body,
      grid=(_NB,),
      in_specs=[
          pl.BlockSpec((_NC, _RB, _D), lambda i: (0, i, 0)),
          pl.BlockSpec((_NC, _RB, _D), lambda i: (0, i, 0)),
          pl.BlockSpec((_RB, _D), lambda i: (i, 0)),
          pl.BlockSpec((_D, _D), lambda i: (0, 0)),
          pl.BlockSpec((1, _D), lambda i: (0, 0)),
          pl.BlockSpec((_D, _D), lambda i: (0, 0)),
      ],
      out_specs=[
          pl.BlockSpec((_RB, _D), lambda i: (i, 0)),
          pl.BlockSpec((2, _D), lambda i: (0, 0)),
      ],
      out_shape=[
          jax.ShapeDtypeStruct((_N, _D), jnp.float32),
          jax.ShapeDtypeStruct((2, _D), jnp.float32),
      ],
  )(p, cnt, x, w_l, b_l, w_r)


def _bn_body(u_ref, st_ref, g_ref, be_ref, o_ref, *, relu):
  mean = st_ref[0:1, :] * (1.0 / _N)
  var = st_ref[1:2, :] * (1.0 / _N) - mean * mean
  scale = lax.rsqrt(var + _EPS) * g_ref[...]
  o = (u_ref[...] - mean) * scale + be_ref[...]
  if relu:
    o = jnp.maximum(o, 0.0)
  o_ref[...] = o


def _batch_norm(u, st, g, be, relu):
  return pl.pallas_call(
      functools.partial(_bn_body, relu=relu),
      grid=(_NB,),
      in_specs=[
          pl.BlockSpec((_RB, _D), lambda i: (i, 0)),
          pl.BlockSpec((2, _D), lambda i: (0, 0)),
          pl.BlockSpec((1, _D), lambda i: (0, 0)),
          pl.BlockSpec((1, _D), lambda i: (0, 0)),
      ],
      out_specs=pl.BlockSpec((_RB, _D), lambda i: (i, 0)),
      out_shape=jax.ShapeDtypeStruct((_N, _D), jnp.float32),
  )(u, st, g, be)


def kernel(x, edge_index, W_l1, b_l1, W_r1, g1, be1, W_l2, b_l2, W_r2, g2, be2):
  ei = edge_index.astype(jnp.int32)
  npad = _E_PAD - _E
  pad_i = jnp.arange(npad, dtype=jnp.int32)
  src = jnp.concatenate([ei[0], pad_i % _N]).reshape(_NW, _NCHUNK, _CHUNK)
  dst = jnp.concatenate([ei[1], _N + (pad_i % 112)]).reshape(_NW, _NCHUNK, _CHUNK)

  zf = jnp.zeros((_N_ACC, _D), jnp.float32)
  ones = jnp.ones((_CHUNK, _D), jnp.float32)

  b_l1 = b_l1.reshape(1, _D)
  b_l2 = b_l2.reshape(1, _D)
  g1 = g1.reshape(1, _D)
  g2 = g2.reshape(1, _D)
  be1 = be1.reshape(1, _D)
  be2 = be2.reshape(1, _D)

  (cnt,) = _count(dst, zf, ones)
  (p1,) = _agg(x, src, dst, zf)
  u1, st1 = _dense(p1, cnt, x, W_l1, b_l1, W_r1)
  h1 = _batch_norm(u1, st1, g1, be1, relu=True)
  (p2,) = _agg(h1, src, dst, zf)
  u2, st2 = _dense(p2, cnt, h1, W_l2, b_l2, W_r2)
  return _batch_norm(u2, st2, g2, be2, relu=False)


# speedup vs baseline: 10.4170x; 1.3625x over previous
import functools

import jax
import jax.numpy as jnp
from jax import lax
from jax.experimental import pallas as pl
from jax.experimental.pallas import tpu as pltpu
from jax.experimental.pallas import tpu_sc as plsc

_N = 10000
_D = 128
_E = 320000
_EPS = 1e-5

_NC = 2
_NS = 16
_NW = _NC * _NS
_CHUNK = 128
_NCHUNK = 80
_E_PAD = _NW * _NCHUNK * _CHUNK
_N_ACC = _N + 112
_RPT = _N_ACC // _NS

_RB = 1000
_NB = _N // _RB


def _mesh():
  return plsc.VectorSubcoreMesh(core_axis_name="c", subcore_axis_name="s")


def _unpack(pkv, j, srcs, dsts, slot):
  for k in range(_CHUNK // 16):
    v = pkv[j, pl.ds(16 * k, 16)]
    srcs[slot, pl.ds(16 * k, 16)] = v & 0xFFFF
    dsts[slot, pl.ds(16 * k, 16)] = lax.shift_right_logical(v, 16)


def _agg_body(x_hbm, pk_hbm, zf_hbm, p_hbm, pkv, srcs, dsts, gbuf, accs,
              sem0, sem1):
  cid = lax.axis_index("c")
  sid = lax.axis_index("s")
  wid = sid * _NC + cid
  r0 = sid * _RPT
  pltpu.sync_copy(zf_hbm.at[pl.ds(r0, _RPT)], accs.at[pl.ds(r0, _RPT)])
  pltpu.sync_copy(pk_hbm.at[wid], pkv)
  plsc.subcore_barrier()

  ga = gbuf.at[0]
  gb = gbuf.at[1]
  _unpack(pkv, 0, srcs, dsts, 0)
  pltpu.async_copy(x_hbm.at[srcs.at[0]], ga, sem0)

  def step(i, carry):
    j0 = 2 * i
    j1 = j0 + 1
    j2 = j0 + 2
    _unpack(pkv, j1, srcs, dsts, 1)
    pltpu.async_copy(x_hbm.at[srcs.at[1]], gb, sem1)
    pltpu.make_async_copy(x_hbm.at[srcs.at[0]], ga, sem0).wait()
    pltpu.sync_copy(ga, accs.at[dsts.at[0]], add=True)

    @pl.when(j2 < _NCHUNK)
    def _():
      _unpack(pkv, j2, srcs, dsts, 0)
      pltpu.async_copy(x_hbm.at[srcs.at[0]], ga, sem0)

    pltpu.make_async_copy(x_hbm.at[srcs.at[1]], gb, sem1).wait()
    pltpu.sync_copy(gb, accs.at[dsts.at[1]], add=True)
    return carry

  lax.fori_loop(0, _NCHUNK // 2, step, 0)
  plsc.subcore_barrier()
  pltpu.sync_copy(accs.at[pl.ds(r0, _RPT)], p_hbm.at[cid, pl.ds(r0, _RPT)])


_agg = pl.kernel(
    _agg_body,
    mesh=_mesh(),
    out_type=[jax.ShapeDtypeStruct((_NC, _N_ACC, _D), jnp.float32)],
    scratch_types=[
        pltpu.VMEM((_NCHUNK, _CHUNK), jnp.int32),
        pltpu.VMEM((2, _CHUNK), jnp.int32),
        pltpu.VMEM((2, _CHUNK), jnp.int32),
        pltpu.VMEM((2, _CHUNK, _D), jnp.float32),
        pltpu.VMEM_SHARED((_N_ACC, _D), jnp.float32),
        pltpu.SemaphoreType.DMA,
        pltpu.SemaphoreType.DMA,
    ],
)


def _count_body(pk_hbm, zf_hbm, ones_hbm, cnt_hbm, pkv, dsts, onesv, cnts):
  cid = lax.axis_index("c")
  sid = lax.axis_index("s")
  wid = sid * _NC + cid
  r0 = sid * _RPT
  pltpu.sync_copy(zf_hbm.at[pl.ds(r0, _RPT)], cnts.at[pl.ds(r0, _RPT)])
  pltpu.sync_copy(ones_hbm, onesv)
  pltpu.sync_copy(pk_hbm.at[wid], pkv)
  plsc.subcore_barrier()

  def step(j, carry):
    for k in range(_CHUNK // 16):
      dsts[0, pl.ds(16 * k, 16)] = lax.shift_right_logical(
          pkv[j, pl.ds(16 * k, 16)], 16)
    pltpu.sync_copy(onesv, cnts.at[dsts.at[0]], add=True)
    return carry

  lax.fori_loop(0, _NCHUNK, step, 0)
  plsc.subcore_barrier()
  pltpu.sync_copy(cnts.at[pl.ds(r0, _RPT)], cnt_hbm.at[cid, pl.ds(r0, _RPT)])


_count = pl.kernel(
    _count_body,
    mesh=_mesh(),
    out_type=[jax.ShapeDtypeStruct((_NC, _N_ACC, _D), jnp.float32)],
    scratch_types=[
        pltpu.VMEM((_NCHUNK, _CHUNK), jnp.int32),
        pltpu.VMEM((1, _CHUNK), jnp.int32),
        pltpu.VMEM((_CHUNK, _D), jnp.float32),
        pltpu.VMEM_SHARED((_N_ACC, _D), jnp.float32),
    ],
)


def _dense_body(p_ref, c_ref, x_ref, wl_ref, bl_ref, wr_ref, u_ref, st_ref):
  agg = p_ref[0] + p_ref[1]
  cnt = jnp.maximum(c_ref[0][:, 0:1] + c_ref[1][:, 0:1], 1.0)
  agg = agg / cnt
  u = (lax.dot_general(agg, wl_ref[...], (((1,), (1,)), ((), ())),
                       preferred_element_type=jnp.float32)
       + bl_ref[...]
       + lax.dot_general(x_ref[...], wr_ref[...], (((1,), (1,)), ((), ())),
                         preferred_element_type=jnp.float32))
  u_ref[...] = u

  @pl.when(pl.program_id(0) == 0)
  def _():
    st_ref[...] = jnp.zeros_like(st_ref)

  st_ref[...] += jnp.concatenate(
      [jnp.sum(u, axis=0, keepdims=True),
       jnp.sum(u * u, axis=0, keepdims=True)], axis=0)


def _dense(p, cnt, x, w_l, b_l, w_r):
  return pl.pallas_call(
      _dense_body,
      grid=(_NB,),
      in_specs=[
          pl.BlockSpec((_NC, _RB, _D), lambda i: (0, i, 0)),
          pl.BlockSpec((_NC, _RB, _D), lambda i: (0, i, 0)),
          pl.BlockSpec((_RB, _D), lambda i: (i, 0)),
          pl.BlockSpec((_D, _D), lambda i: (0, 0)),
          pl.BlockSpec((1, _D), lambda i: (0, 0)),
          pl.BlockSpec((_D, _D), lambda i: (0, 0)),
      ],
      out_specs=[
          pl.BlockSpec((_RB, _D), lambda i: (i, 0)),
          pl.BlockSpec((2, _D), lambda i: (0, 0)),
      ],
      out_shape=[
          jax.ShapeDtypeStruct((_N, _D), jnp.float32),
          jax.ShapeDtypeStruct((2, _D), jnp.float32),
      ],
  )(p, cnt, x, w_l, b_l, w_r)


def _bn_body(u_ref, st_ref, g_ref, be_ref, o_ref, *, relu):
  mean = st_ref[0:1, :] * (1.0 / _N)
  var = st_ref[1:2, :] * (1.0 / _N) - mean * mean
  scale = lax.rsqrt(var + _EPS) * g_ref[...]
  o = (u_ref[...] - mean) * scale + be_ref[...]
  if relu:
    o = jnp.maximum(o, 0.0)
  o_ref[...] = o


def _batch_norm(u, st, g, be, relu):
  return pl.pallas_call(
      functools.partial(_bn_body, relu=relu),
      grid=(_NB,),
      in_specs=[
          pl.BlockSpec((_RB, _D), lambda i: (i, 0)),
          pl.BlockSpec((2, _D), lambda i: (0, 0)),
          pl.BlockSpec((1, _D), lambda i: (0, 0)),
          pl.BlockSpec((1, _D), lambda i: (0, 0)),
      ],
      out_specs=pl.BlockSpec((_RB, _D), lambda i: (i, 0)),
      out_shape=jax.ShapeDtypeStruct((_N, _D), jnp.float32),
  )(u, st, g, be)


def kernel(x, edge_index, W_l1, b_l1, W_r1, g1, be1, W_l2, b_l2, W_r2, g2, be2):
  ei = edge_index.astype(jnp.int32)
  npad = _E_PAD - _E
  pad_i = jnp.arange(npad, dtype=jnp.int32)
  src = jnp.concatenate([ei[0], pad_i % _N])
  dst = jnp.concatenate([ei[1], _N + (pad_i % 112)])
  pk = (src | (dst << 16)).reshape(_NW, _NCHUNK, _CHUNK)

  zf = jnp.zeros((_N_ACC, _D), jnp.float32)
  ones = jnp.ones((_CHUNK, _D), jnp.float32)

  b_l1 = b_l1.reshape(1, _D)
  b_l2 = b_l2.reshape(1, _D)
  g1 = g1.reshape(1, _D)
  g2 = g2.reshape(1, _D)
  be1 = be1.reshape(1, _D)
  be2 = be2.reshape(1, _D)

  (cnt,) = _count(pk, zf, ones)
  (p1,) = _agg(x, pk, zf)
  u1, st1 = _dense(p1, cnt, x, W_l1, b_l1, W_r1)
  h1 = _batch_norm(u1, st1, g1, be1, relu=True)
  (p2,) = _agg(h1, pk, zf)
  u2, st2 = _dense(p2, cnt, h1, W_l2, b_l2, W_r2)
  return _batch_norm(u2, st2, g2, be2, relu=False)


# speedup vs baseline: 12.1705x; 1.1683x over previous
import functools

import jax
import jax.numpy as jnp
from jax import lax
from jax.experimental import pallas as pl
from jax.experimental.pallas import tpu as pltpu
from jax.experimental.pallas import tpu_sc as plsc

_N = 10000
_D = 128
_E = 320000
_EPS = 1e-5

_NC = 2
_NS = 16
_NW = _NC * _NS
_CHUNK = 128
_NCHUNK = 80
_E_PAD = _NW * _NCHUNK * _CHUNK
_N_ACC = _N + 112
_RPT = _N_ACC // _NS

_RB = 1000
_NB = _N // _RB


def _mesh():
  return plsc.VectorSubcoreMesh(core_axis_name="c", subcore_axis_name="s")


def _unpack(pkv, j, srcs, dsts, slot):
  for k in range(_CHUNK // 16):
    v = pkv[j, pl.ds(16 * k, 16)]
    srcs[slot, pl.ds(16 * k, 16)] = v & 0xFFFF
    dsts[slot, pl.ds(16 * k, 16)] = lax.shift_right_logical(v, 16)


def _agg_body(x_hbm, pk_hbm, zf_hbm, p_hbm, pkv, srcs, dsts, gbuf, accs,
              sem0, sem1):
  cid = lax.axis_index("c")
  sid = lax.axis_index("s")
  wid = sid * _NC + cid
  r0 = sid * _RPT
  pltpu.sync_copy(zf_hbm.at[pl.ds(r0, _RPT)], accs.at[pl.ds(r0, _RPT)])
  pltpu.sync_copy(pk_hbm.at[wid], pkv)
  plsc.subcore_barrier()

  ga = gbuf.at[0]
  gb = gbuf.at[1]
  _unpack(pkv, 0, srcs, dsts, 0)
  pltpu.async_copy(x_hbm.at[srcs.at[0]], ga, sem0)

  def step(i, carry):
    j0 = 2 * i
    j1 = j0 + 1
    j2 = j0 + 2
    _unpack(pkv, j1, srcs, dsts, 1)
    pltpu.async_copy(x_hbm.at[srcs.at[1]], gb, sem1)
    pltpu.make_async_copy(x_hbm.at[srcs.at[0]], ga, sem0).wait()
    pltpu.sync_copy(ga, accs.at[dsts.at[0]], add=True)

    @pl.when(j2 < _NCHUNK)
    def _():
      _unpack(pkv, j2, srcs, dsts, 0)
      pltpu.async_copy(x_hbm.at[srcs.at[0]], ga, sem0)

    pltpu.make_async_copy(x_hbm.at[srcs.at[1]], gb, sem1).wait()
    pltpu.sync_copy(gb, accs.at[dsts.at[1]], add=True)
    return carry

  lax.fori_loop(0, _NCHUNK // 2, step, 0)
  plsc.subcore_barrier()
  pltpu.sync_copy(accs.at[pl.ds(r0, _RPT)], p_hbm.at[cid, pl.ds(r0, _RPT)])


_agg = pl.kernel(
    _agg_body,
    mesh=_mesh(),
    out_type=[jax.ShapeDtypeStruct((_NC, _N_ACC, _D), jnp.float32)],
    scratch_types=[
        pltpu.VMEM((_NCHUNK, _CHUNK), jnp.int32),
        pltpu.VMEM((2, _CHUNK), jnp.int32),
        pltpu.VMEM((2, _CHUNK), jnp.int32),
        pltpu.VMEM((2, _CHUNK, _D), jnp.float32),
        pltpu.VMEM_SHARED((_N_ACC, _D), jnp.float32),
        pltpu.SemaphoreType.DMA,
        pltpu.SemaphoreType.DMA,
    ],
)


_CW = 16
_ZR = _RPT // 8


def _count_body(pk_hbm, cnt_hbm, pkv, dsts, onesv, zbuf, cnts):
  cid = lax.axis_index("c")
  sid = lax.axis_index("s")
  wid = sid * _NC + cid
  r0 = sid * _RPT
  for r in range(_CHUNK):
    onesv[r] = jnp.full((_CW,), 1.0, jnp.float32)
  for r in range(_ZR):
    zbuf[r] = jnp.zeros((_CW,), jnp.float32)
  for m in range(8):
    pltpu.sync_copy(zbuf, cnts.at[pl.ds(r0 + m * _ZR, _ZR)])
  pltpu.sync_copy(pk_hbm.at[wid], pkv)
  plsc.subcore_barrier()

  def step(j, carry):
    for k in range(_CHUNK // 16):
      dsts[0, pl.ds(16 * k, 16)] = lax.shift_right_logical(
          pkv[j, pl.ds(16 * k, 16)], 16)
    pltpu.sync_copy(onesv, cnts.at[dsts.at[0]], add=True)
    return carry

  lax.fori_loop(0, _NCHUNK, step, 0)
  plsc.subcore_barrier()
  pltpu.sync_copy(cnts.at[pl.ds(r0, _RPT)], cnt_hbm.at[cid, pl.ds(r0, _RPT)])


_count = pl.kernel(
    _count_body,
    mesh=_mesh(),
    out_type=[jax.ShapeDtypeStruct((_NC, _N_ACC, _CW), jnp.float32)],
    scratch_types=[
        pltpu.VMEM((_NCHUNK, _CHUNK), jnp.int32),
        pltpu.VMEM((1, _CHUNK), jnp.int32),
        pltpu.VMEM((_CHUNK, _CW), jnp.float32),
        pltpu.VMEM((_ZR, _CW), jnp.float32),
        pltpu.VMEM_SHARED((_N_ACC, _CW), jnp.float32),
    ],
    compiler_params=pltpu.CompilerParams(use_tc_tiling_on_sc=False),
)


def _rmat_body(x_ref, wr_ref, bl_ref, r_ref):
  r_ref[...] = lax.dot_general(
      x_ref[...], wr_ref[...], (((1,), (1,)), ((), ())),
      preferred_element_type=jnp.float32) + bl_ref[...]


def _rmat(x, w_r, b_l):
  return pl.pallas_call(
      _rmat_body,
      grid=(_NB,),
      in_specs=[
          pl.BlockSpec((_RB, _D), lambda i: (i, 0)),
          pl.BlockSpec((_D, _D), lambda i: (0, 0)),
          pl.BlockSpec((1, _D), lambda i: (0, 0)),
      ],
      out_specs=pl.BlockSpec((_RB, _D), lambda i: (i, 0)),
      out_shape=jax.ShapeDtypeStruct((_N, _D), jnp.float32),
  )(x, w_r, b_l)


def _dense_body(p_ref, c_ref, r_ref, wl_ref, u_ref, st_ref):
  agg = p_ref[0] + p_ref[1]
  cnt = jnp.maximum(c_ref[0] + c_ref[1], 1.0)
  agg = agg / cnt
  u = lax.dot_general(agg, wl_ref[...], (((1,), (1,)), ((), ())),
                      preferred_element_type=jnp.float32) + r_ref[...]
  u_ref[...] = u

  @pl.when(pl.program_id(0) == 0)
  def _():
    st_ref[...] = jnp.zeros_like(st_ref)

  st_ref[...] += jnp.concatenate(
      [jnp.sum(u, axis=0, keepdims=True),
       jnp.sum(u * u, axis=0, keepdims=True)], axis=0)


def _dense(p, cnt, r, w_l):
  return pl.pallas_call(
      _dense_body,
      grid=(_NB,),
      in_specs=[
          pl.BlockSpec((_NC, _RB, _D), lambda i: (0, i, 0)),
          pl.BlockSpec((_NC, _RB, 1), lambda i: (0, i, 0)),
          pl.BlockSpec((_RB, _D), lambda i: (i, 0)),
          pl.BlockSpec((_D, _D), lambda i: (0, 0)),
      ],
      out_specs=[
          pl.BlockSpec((_RB, _D), lambda i: (i, 0)),
          pl.BlockSpec((2, _D), lambda i: (0, 0)),
      ],
      out_shape=[
          jax.ShapeDtypeStruct((_N, _D), jnp.float32),
          jax.ShapeDtypeStruct((2, _D), jnp.float32),
      ],
  )(p, cnt, r, w_l)


def _bn_body(u_ref, st_ref, g_ref, be_ref, o_ref, *, relu):
  mean = st_ref[0:1, :] * (1.0 / _N)
  var = st_ref[1:2, :] * (1.0 / _N) - mean * mean
  scale = lax.rsqrt(var + _EPS) * g_ref[...]
  o = (u_ref[...] - mean) * scale + be_ref[...]
  if relu:
    o = jnp.maximum(o, 0.0)
  o_ref[...] = o


def _batch_norm(u, st, g, be, relu):
  return pl.pallas_call(
      functools.partial(_bn_body, relu=relu),
      grid=(_NB,),
      in_specs=[
          pl.BlockSpec((_RB, _D), lambda i: (i, 0)),
          pl.BlockSpec((2, _D), lambda i: (0, 0)),
          pl.BlockSpec((1, _D), lambda i: (0, 0)),
          pl.BlockSpec((1, _D), lambda i: (0, 0)),
      ],
      out_specs=pl.BlockSpec((_RB, _D), lambda i: (i, 0)),
      out_shape=jax.ShapeDtypeStruct((_N, _D), jnp.float32),
  )(u, st, g, be)


def kernel(x, edge_index, W_l1, b_l1, W_r1, g1, be1, W_l2, b_l2, W_r2, g2, be2):
  ei = edge_index.astype(jnp.int32)
  npad = _E_PAD - _E
  pad_i = jnp.arange(npad, dtype=jnp.int32)
  src = jnp.concatenate([ei[0], pad_i % _N])
  dst = jnp.concatenate([ei[1], _N + (pad_i % 112)])
  pk = (src | (dst << 16)).reshape(_NW, _NCHUNK, _CHUNK)

  zf = jnp.zeros((_N_ACC, _D), jnp.float32)

  b_l1 = b_l1.reshape(1, _D)
  b_l2 = b_l2.reshape(1, _D)
  g1 = g1.reshape(1, _D)
  g2 = g2.reshape(1, _D)
  be1 = be1.reshape(1, _D)
  be2 = be2.reshape(1, _D)

  (cntw,) = _count(pk)
  cnt = cntw[:, :_N, :1]
  (p1,) = _agg(x, pk, zf)
  r1 = _rmat(x, W_r1, b_l1)
  u1, st1 = _dense(p1, cnt, r1, W_l1)
  h1 = _batch_norm(u1, st1, g1, be1, relu=True)
  (p2,) = _agg(h1, pk, zf)
  r2 = _rmat(h1, W_r2, b_l2)
  u2, st2 = _dense(p2, cnt, r2, W_l2)
  return _batch_norm(u2, st2, g2, be2, relu=False)
